# speedup vs baseline: 45.7952x; 45.7952x over previous
import functools

import numpy as np
import jax
import jax.numpy as jnp
from jax.experimental import pallas as pl

_INT_MIN = np.int32(-2147483648)
_INV_SQRT2 = np.float32(1.0 / np.sqrt(2.0))


def _router_kernel(x_ref, w1_ref, b1_ref, w2_ref, b2_ref, scores_ref, mask_ref,
                   *, k):
    h = jnp.dot(x_ref[...], w1_ref[...], preferred_element_type=jnp.float32)
    h = h + b1_ref[...]
    h = 0.5 * h * (1.0 + jax.lax.erf(h * _INV_SQRT2))
    s = jnp.dot(h, w2_ref[...], preferred_element_type=jnp.float32)
    s = s + b2_ref[...]
    scores_ref[...] = s

    si = jax.lax.bitcast_convert_type(s, jnp.int32)
    w = jnp.where(si < 0, si ^ jnp.int32(0x7FFFFFFF), si)

    bm = s.shape[0]

    def body(i, c):
        bit_val = jnp.left_shift(jnp.int32(1), jnp.int32(31) - i)
        cand_off = c | bit_val
        cand = cand_off ^ _INT_MIN
        cnt = jnp.sum((w >= cand).astype(jnp.int32), axis=1, keepdims=True)
        return jnp.where(cnt >= k, cand_off, c)

    c = jax.lax.fori_loop(0, 32, body, jnp.zeros((bm, 1), jnp.int32))
    thresh = c ^ _INT_MIN
    mask_ref[...] = (w >= thresh).astype(jnp.float32)


def kernel(hidden_state, W1, b1, W2, b2):
    B, S, H = hidden_state.shape
    BOT = W1.shape[1]
    F = W2.shape[1]
    M = B * S
    k = max(1, int(F * (1.0 - 0.9)))

    BM = 256
    grid = (M // BM,)

    x = hidden_state.reshape(M, H)
    scores, mask = pl.pallas_call(
        functools.partial(_router_kernel, k=k),
        grid=grid,
        in_specs=[
            pl.BlockSpec((BM, H), lambda i: (i, 0)),
            pl.BlockSpec((H, BOT), lambda i: (0, 0)),
            pl.BlockSpec((1, BOT), lambda i: (0, 0)),
            pl.BlockSpec((BOT, F), lambda i: (0, 0)),
            pl.BlockSpec((1, F), lambda i: (0, 0)),
        ],
        out_specs=[
            pl.BlockSpec((BM, F), lambda i: (i, 0)),
            pl.BlockSpec((BM, F), lambda i: (i, 0)),
        ],
        out_shape=[
            jax.ShapeDtypeStruct((M, F), jnp.float32),
            jax.ShapeDtypeStruct((M, F), jnp.float32),
        ],
    )(x, W1, b1.reshape(1, BOT), W2, b2.reshape(1, F))
    return scores.reshape(B, S, F), mask.reshape(B, S, F)


# speedup vs baseline: 64.8007x; 1.4150x over previous
import functools

import numpy as np
import jax
import jax.numpy as jnp
from jax.experimental import pallas as pl

_INT_MIN = np.int32(-2147483648)
_INV_SQRT2 = np.float32(1.0 / np.sqrt(2.0))


def _router_kernel(x_ref, w1_ref, b1_ref, w2_ref, b2_ref, scores_ref, mask_ref,
                   *, k):
    h = jnp.dot(x_ref[...], w1_ref[...], preferred_element_type=jnp.float32)
    h = h + b1_ref[...]
    h = 0.5 * h * (1.0 + jax.lax.erf(h * _INV_SQRT2))
    s = jnp.dot(h, w2_ref[...], preferred_element_type=jnp.float32)
    s = s + b2_ref[...]
    scores_ref[...] = s

    si = jax.lax.bitcast_convert_type(s, jnp.int32)
    w = jnp.where(si < 0, si ^ jnp.int32(0x7FFFFFFF), si)

    bm = s.shape[0]
    i16_min = jnp.int16(-32768)

    def row_count16(pred):
        t = jnp.where(pred, jnp.int16(1), jnp.int16(0))
        width = t.shape[1]
        while width > 512:
            half = width // 2
            t = t[:, :half] + t[:, half:]
            width = half
        return jnp.sum(t.astype(jnp.int32), axis=1, keepdims=True)

    w16 = jnp.right_shift(w, 16).astype(jnp.int16)

    def body_hi(i, c):
        bit_val = jnp.left_shift(jnp.int32(1), jnp.int32(15) - i)
        cand_off = c | bit_val
        cand = (cand_off ^ jnp.int32(32768)).astype(jnp.int16)
        cnt = row_count16(w16 >= cand)
        return jnp.where(cnt >= k, cand_off, c)

    c_hi_off = jax.lax.fori_loop(0, 16, body_hi,
                                 jnp.zeros((bm, 1), jnp.int32))
    p16 = (c_hi_off ^ jnp.int32(32768)).astype(jnp.int16)

    n_above = row_count16(w16 > p16)

    l16 = jnp.where(w16 == p16, w.astype(jnp.int16) ^ i16_min, i16_min)

    def body_lo(i, c):
        bit_val = jnp.left_shift(jnp.int32(1), jnp.int32(15) - i)
        cand_off = c | bit_val
        cand = (cand_off ^ jnp.int32(32768)).astype(jnp.int16)
        cnt = row_count16(l16 >= cand)
        return jnp.where(n_above + cnt >= k, cand_off, c)

    c_lo_off = jax.lax.fori_loop(0, 16, body_lo,
                                 jnp.zeros((bm, 1), jnp.int32))

    thresh = (jnp.left_shift(c_hi_off - jnp.int32(32768), 16)
              | (c_lo_off & jnp.int32(0xFFFF)))
    mask_ref[...] = (w >= thresh).astype(jnp.float32)


def kernel(hidden_state, W1, b1, W2, b2):
    B, S, H = hidden_state.shape
    BOT = W1.shape[1]
    F = W2.shape[1]
    M = B * S
    k = max(1, int(F * (1.0 - 0.9)))

    BM = 256
    grid = (M // BM,)

    x = hidden_state.reshape(M, H)
    scores, mask = pl.pallas_call(
        functools.partial(_router_kernel, k=k),
        grid=grid,
        in_specs=[
            pl.BlockSpec((BM, H), lambda i: (i, 0)),
            pl.BlockSpec((H, BOT), lambda i: (0, 0)),
            pl.BlockSpec((1, BOT), lambda i: (0, 0)),
            pl.BlockSpec((BOT, F), lambda i: (0, 0)),
            pl.BlockSpec((1, F), lambda i: (0, 0)),
        ],
        out_specs=[
            pl.BlockSpec((BM, F), lambda i: (i, 0)),
            pl.BlockSpec((BM, F), lambda i: (i, 0)),
        ],
        out_shape=[
            jax.ShapeDtypeStruct((M, F), jnp.float32),
            jax.ShapeDtypeStruct((M, F), jnp.float32),
        ],
    )(x, W1, b1.reshape(1, BOT), W2, b2.reshape(1, F))
    return scores.reshape(B, S, F), mask.reshape(B, S, F)
